# Initial kernel scaffold; baseline (speedup 1.0000x reference)
import jax
import jax.numpy as jnp
from jax.experimental import pallas as pl


def kernel(head_ids, rel_ids, tail_ids, entity_table, relation_table):
    raise NotImplementedError("write your pallas kernel here")


# speedup vs baseline: 2.5598x; 2.5598x over previous
import functools

import jax
import jax.numpy as jnp
from jax.experimental import pallas as pl
from jax.experimental.pallas import tpu as pltpu

B = 16384
D = 128
R = 500
BLK = 128
NPAD = B + BLK
NW_OUT = NPAD // BLK


def _tc_body(offs_ref, m_ref, h_ref, t_ref, out_ref):
    r = pl.program_id(0)

    @pl.when(r == 0)
    def _init():
        out_ref[...] = jnp.zeros_like(out_ref)

    off0 = offs_ref[r]
    off1 = offs_ref[r + 1]
    w0 = off0 // BLK
    nw = jnp.where(off1 > off0, (off1 - w0 * BLK + BLK - 1) // BLK, 0)
    m = m_ref[0]

    def body(k, _):
        w = w0 + k
        base = w * BLK
        hs = h_ref[pl.ds(base, BLK), :]
        ts = t_ref[pl.ds(base, BLK), :]
        proj = jax.lax.dot_general(
            hs, m, (((1,), (0,)), ((), ())),
            preferred_element_type=jnp.float32,
            precision=jax.lax.Precision.HIGHEST,
        )
        s = jnp.sum(proj * ts, axis=1)
        idx = base + jax.lax.broadcasted_iota(jnp.int32, (BLK,), 0)
        contrib = jnp.where((idx >= off0) & (idx < off1), s, 0.0)
        out_ref[pl.ds(base, BLK)] = out_ref[pl.ds(base, BLK)] + contrib
        return 0

    jax.lax.fori_loop(0, nw, body, 0)


def _grouped_scores(offs, m3, h_sorted_pad, t_sorted_pad):
    grid_spec = pltpu.PrefetchScalarGridSpec(
        num_scalar_prefetch=1,
        grid=(R,),
        in_specs=[
            pl.BlockSpec((1, D, D), lambda r, offs: (r, 0, 0)),
            pl.BlockSpec((NPAD, D), lambda r, offs: (0, 0)),
            pl.BlockSpec((NPAD, D), lambda r, offs: (0, 0)),
        ],
        out_specs=pl.BlockSpec((NPAD,), lambda r, offs: (0,)),
    )
    return pl.pallas_call(
        _tc_body,
        grid_spec=grid_spec,
        out_shape=jax.ShapeDtypeStruct((NPAD,), jnp.float32),
        compiler_params=pltpu.CompilerParams(
            dimension_semantics=("arbitrary",),
        ),
    )(offs, m3, h_sorted_pad, t_sorted_pad)


def kernel(head_ids, rel_ids, tail_ids, entity_table, relation_table):
    iota = jnp.arange(B, dtype=jnp.int32)
    skey = jnp.sort(rel_ids.astype(jnp.int32) * 32768 + iota)
    perm = skey & 32767
    srel = skey >> 15
    offs = jnp.searchsorted(srel, jnp.arange(R + 1, dtype=jnp.int32)).astype(jnp.int32)
    sorted_head = jnp.take(head_ids, perm)
    sorted_tail = jnp.take(tail_ids, perm)
    invperm = jnp.zeros((B,), jnp.int32).at[perm].set(iota)

    h_sorted = jnp.take(entity_table, sorted_head, axis=0)
    t_sorted = jnp.take(entity_table, sorted_tail, axis=0)
    h_pad = jnp.pad(h_sorted, ((0, NPAD - B), (0, 0)))
    t_pad = jnp.pad(t_sorted, ((0, NPAD - B), (0, 0)))

    m3 = relation_table.reshape(R, D, D)
    scores_sorted = _grouped_scores(offs, m3, h_pad, t_pad)[:B]

    return jnp.take(scores_sorted, invperm)
